# baseline (speedup 1.0000x reference)
import jax
import jax.numpy as jnp
from jax import lax
from jax.experimental import pallas as pl
from jax.experimental.pallas import tpu as pltpu
from jax.experimental.pallas import tpu_sc as plsc

N, C, K, M, OUT = 10000, 128, 16, 4, 128
NUM_WORKERS = 32
NB = 320
NPAD = NUM_WORKERS * NB
TPAD = 10016
L = 16
CCH = C // L


SB = 8
NBLK = NB // SB
CE = 144


def _sc_aggregate(x_hbm, adj_hbm, uxc_hbm, y_hbm,
                  uxc_v, adj_v, rows0_v, rows1_v, st0_v, st1_v,
                  gsem0, gsem1, osem0, osem1):
    nc = plsc.get_sparse_core_info().num_cores
    wid = lax.axis_index("s") * nc + lax.axis_index("c")
    base = wid * NB
    pltpu.sync_copy(adj_hbm.at[pl.ds(wid * NBLK, NBLK)], adj_v)
    pltpu.sync_copy(uxc_hbm.at[pl.ds(base, NB)], uxc_v)

    midx = [jnp.full((L,), m, jnp.int32) for m in range(M)]
    cidx = [jnp.full((L,), C + m, jnp.int32) for m in range(M)]
    kiota = lax.iota(jnp.int32, L)

    def process(jb, rows_ref, st_ref):
        def node(s, carry):
            adj_vec = adj_v[jb, pl.ds(s * K, K)]
            nid = jnp.full((L,), jb * SB + s, jnp.int32)
            kidx = kiota + s * K
            nb_logit = [plsc.load_gather(rows_ref, [kidx, cidx[m]])
                        for m in range(M)]
            own = [plsc.load_gather(uxc_v, [nid, midx[m]]) for m in range(M)]
            p = [own[m] - nb_logit[m] for m in range(M)]
            pmax = jnp.maximum(jnp.maximum(p[0], p[1]),
                               jnp.maximum(p[2], p[3]))
            e = [jnp.exp(pm - pmax) for pm in p]
            ssum = e[0] + e[1] + e[2] + e[3]
            cnt = plsc.all_reduce_population_count(adj_vec != 0)
            inv = jnp.where(cnt > 0, 1.0 / cnt.astype(jnp.float32), 0.0)
            scale = inv / ssum
            wv = [e[m] * scale for m in range(M)]
            acc = [[jnp.zeros((L,), jnp.float32) for _ in range(CCH)]
                   for _ in range(M)]
            for k in range(K):
                r = [rows_ref[s * K + k, cc * L:(cc + 1) * L]
                     for cc in range(CCH)]
                for m in range(M):
                    wk = wv[m][k]
                    for cc in range(CCH):
                        acc[m][cc] = acc[m][cc] + wk * r[cc]
            for m in range(M):
                for cc in range(CCH):
                    st_ref[s, m * C + cc * L:m * C + (cc + 1) * L] = \
                        acc[m][cc]
            return carry

        lax.fori_loop(0, SB, node, 0)

    def flush(jb, st_ref, osem):
        pltpu.async_copy(st_ref, y_hbm.at[pl.ds(base + jb * SB, SB)], osem)

    def gather(jb, rows_ref, gsem):
        pltpu.async_copy(x_hbm.at[adj_v.at[jb]], rows_ref, gsem)

    gather(0, rows0_v, gsem0)

    def body(t, carry):
        jb0 = 2 * t
        jb1 = 2 * t + 1
        gather(jb1, rows1_v, gsem1)
        pltpu.make_async_copy(x_hbm.at[adj_v.at[jb0]], rows0_v, gsem0).wait()

        @pl.when(t > 0)
        def _():
            pltpu.make_async_copy(
                st0_v, y_hbm.at[pl.ds(base, SB)], osem0).wait()

        process(jb0, rows0_v, st0_v)
        flush(jb0, st0_v, osem0)
        gather(jnp.minimum(jb0 + 2, NBLK - 1), rows0_v, gsem0)
        pltpu.make_async_copy(x_hbm.at[adj_v.at[jb1]], rows1_v, gsem1).wait()

        @pl.when(t > 0)
        def _():
            pltpu.make_async_copy(
                st1_v, y_hbm.at[pl.ds(base, SB)], osem1).wait()

        process(jb1, rows1_v, st1_v)
        flush(jb1, st1_v, osem1)
        return carry

    lax.fori_loop(0, NBLK // 2, body, 0)
    pltpu.make_async_copy(x_hbm.at[adj_v.at[NBLK - 1]], rows0_v, gsem0).wait()
    pltpu.make_async_copy(st0_v, y_hbm.at[pl.ds(base, SB)], osem0).wait()
    pltpu.make_async_copy(st1_v, y_hbm.at[pl.ds(base, SB)], osem1).wait()


_sc_call = pl.kernel(
    _sc_aggregate,
    out_type=jax.ShapeDtypeStruct((NPAD, M * C), jnp.float32),
    mesh=plsc.VectorSubcoreMesh(core_axis_name="c", subcore_axis_name="s"),
    scratch_types=[
        pltpu.VMEM((NB, M), jnp.float32),
        pltpu.VMEM((NBLK, SB * K), jnp.int32),
        pltpu.VMEM((SB * K, CE), jnp.float32),
        pltpu.VMEM((SB * K, CE), jnp.float32),
        pltpu.VMEM((SB, M * C), jnp.float32),
        pltpu.VMEM((SB, M * C), jnp.float32),
        pltpu.SemaphoreType.DMA,
        pltpu.SemaphoreType.DMA,
        pltpu.SemaphoreType.DMA,
        pltpu.SemaphoreType.DMA,
    ],
    compiler_params=pltpu.CompilerParams(
        needs_layout_passes=False, use_tc_tiling_on_sc=False),
)


def _mm_body(a_ref, w_ref, o_ref):
    o_ref[...] = jnp.dot(a_ref[...], w_ref[...],
                         preferred_element_type=jnp.float32)


def _mm_bias_body(a_ref, w_ref, b_ref, o_ref):
    o_ref[...] = jnp.dot(a_ref[...], w_ref[...],
                         preferred_element_type=jnp.float32) + b_ref[...]


def _matmul(a, w, blk):
    n, kk = a.shape
    _, out = w.shape
    return pl.pallas_call(
        _mm_body,
        grid=(n // blk,),
        in_specs=[
            pl.BlockSpec((blk, kk), lambda i: (i, 0)),
            pl.BlockSpec((kk, out), lambda i: (0, 0)),
        ],
        out_specs=pl.BlockSpec((blk, out), lambda i: (i, 0)),
        out_shape=jax.ShapeDtypeStruct((n, out), jnp.float32),
    )(a, w)


def _matmul_bias(a, w, bias, blk):
    n, kk = a.shape
    _, out = w.shape
    return pl.pallas_call(
        _mm_bias_body,
        grid=(n // blk,),
        in_specs=[
            pl.BlockSpec((blk, kk), lambda i: (i, 0)),
            pl.BlockSpec((kk, out), lambda i: (0, 0)),
            pl.BlockSpec((1, out), lambda i: (0, 0)),
        ],
        out_specs=pl.BlockSpec((blk, out), lambda i: (i, 0)),
        out_shape=jax.ShapeDtypeStruct((n, out), jnp.float32),
    )(a, w, bias)


def kernel(x, adj, W, b, u, c):
    u_padT = jnp.zeros((C, 128), jnp.float32).at[:, :M].set(u.T)
    ux = _matmul(x, u_padT, blk=2000)[:, :M]
    x_tab = (jnp.zeros((TPAD, CE), jnp.float32)
             .at[1:N + 1, :C].set(x)
             .at[1:N + 1, C:C + M].set(ux))
    uxc_pad = jnp.zeros((NPAD, M), jnp.float32).at[:N].set(ux + c[None, :])
    adj_pad = jnp.zeros((NPAD, K), jnp.int32).at[:N].set(adj)
    adj_blk = adj_pad.reshape(NPAD // SB, SB * K)
    y = _sc_call(x_tab, adj_blk, uxc_pad)
    Wcat = jnp.transpose(W, (0, 2, 1)).reshape(M * C, OUT)
    return _matmul_bias(y[:N], Wcat, b.reshape(1, OUT), blk=2000)


# speedup vs baseline: 1.4267x; 1.4267x over previous
import jax
import jax.numpy as jnp
from jax import lax
from jax.experimental import pallas as pl
from jax.experimental.pallas import tpu as pltpu
from jax.experimental.pallas import tpu_sc as plsc

N, C, K, M, OUT = 10000, 128, 16, 4, 128
NUM_WORKERS = 32
NB = 320
NPAD = NUM_WORKERS * NB
TPAD = 10016
L = 16
CCH = C // L


SB = 8
NBLK = NB // SB
LB = 2 * L
CHB = C // LB


def _sc_aggregate(x_hbm, adj_hbm, uxt_hbm, uxc_hbm, y_hbm,
                  ux_v, uxc_v, adj_v, rows0_v, rows1_v, st0_v, st1_v,
                  gsem0, gsem1, osem0, osem1):
    nc = plsc.get_sparse_core_info().num_cores
    wid = lax.axis_index("s") * nc + lax.axis_index("c")
    base = wid * NB
    pltpu.sync_copy(uxt_hbm, ux_v)
    pltpu.sync_copy(adj_hbm.at[pl.ds(wid * NBLK, NBLK)], adj_v)
    pltpu.sync_copy(uxc_hbm.at[pl.ds(base, NB)], uxc_v)

    midx = [jnp.full((L,), m, jnp.int32) for m in range(M)]

    def process(jb, rows_ref, st_ref):
        def node(s, carry):
            adj_vec = adj_v[jb, pl.ds(s * K, K)]
            nid = jnp.full((L,), jb * SB + s, jnp.int32)
            nb_logit = [plsc.load_gather(ux_v, [adj_vec, midx[m]])
                        for m in range(M)]
            own = [plsc.load_gather(uxc_v, [nid, midx[m]]) for m in range(M)]
            p = [own[m] - nb_logit[m] for m in range(M)]
            pmax = jnp.maximum(jnp.maximum(p[0], p[1]),
                               jnp.maximum(p[2], p[3]))
            e = [jnp.exp(pm - pmax) for pm in p]
            ssum = e[0] + e[1] + e[2] + e[3]
            cnt = plsc.all_reduce_population_count(adj_vec != 0)
            inv = jnp.where(cnt > 0, 1.0 / cnt.astype(jnp.float32), 0.0)
            scale = inv / ssum
            wv = [e[m] * scale for m in range(M)]
            acc = [[jnp.zeros((LB,), jnp.bfloat16) for _ in range(CHB)]
                   for _ in range(M)]
            for k in range(K):
                r = [rows_ref[s * K + k, cc * LB:(cc + 1) * LB]
                     for cc in range(CHB)]
                for m in range(M):
                    wk16 = jnp.full((L,), wv[m][k], jnp.float32)
                    wkb = plsc.pack(wk16, wk16,
                                    format=plsc.PackFormat.INTERLEAVED)
                    for cc in range(CHB):
                        acc[m][cc] = acc[m][cc] + wkb * r[cc]
            for m in range(M):
                for cc in range(CHB):
                    st_ref[s, m * C + cc * LB:m * C + (cc + 1) * LB] = \
                        acc[m][cc]
            return carry

        lax.fori_loop(0, SB, node, 0)

    def flush(jb, st_ref, osem):
        pltpu.async_copy(st_ref, y_hbm.at[pl.ds(base + jb * SB, SB)], osem)

    def gather(jb, rows_ref, gsem):
        pltpu.async_copy(x_hbm.at[adj_v.at[jb]], rows_ref, gsem)

    gather(0, rows0_v, gsem0)

    def body(t, carry):
        jb0 = 2 * t
        jb1 = 2 * t + 1
        gather(jb1, rows1_v, gsem1)
        pltpu.make_async_copy(x_hbm.at[adj_v.at[jb0]], rows0_v, gsem0).wait()

        @pl.when(t > 0)
        def _():
            pltpu.make_async_copy(
                st0_v, y_hbm.at[pl.ds(base, SB)], osem0).wait()

        process(jb0, rows0_v, st0_v)
        flush(jb0, st0_v, osem0)
        gather(jnp.minimum(jb0 + 2, NBLK - 1), rows0_v, gsem0)
        pltpu.make_async_copy(x_hbm.at[adj_v.at[jb1]], rows1_v, gsem1).wait()

        @pl.when(t > 0)
        def _():
            pltpu.make_async_copy(
                st1_v, y_hbm.at[pl.ds(base, SB)], osem1).wait()

        process(jb1, rows1_v, st1_v)
        flush(jb1, st1_v, osem1)
        return carry

    lax.fori_loop(0, NBLK // 2, body, 0)
    pltpu.make_async_copy(x_hbm.at[adj_v.at[NBLK - 1]], rows0_v, gsem0).wait()
    pltpu.make_async_copy(st0_v, y_hbm.at[pl.ds(base, SB)], osem0).wait()
    pltpu.make_async_copy(st1_v, y_hbm.at[pl.ds(base, SB)], osem1).wait()


_sc_call = pl.kernel(
    _sc_aggregate,
    out_type=jax.ShapeDtypeStruct((NPAD, M * C), jnp.bfloat16),
    mesh=plsc.VectorSubcoreMesh(core_axis_name="c", subcore_axis_name="s"),
    scratch_types=[
        pltpu.VMEM((TPAD, M), jnp.float32),
        pltpu.VMEM((NB, M), jnp.float32),
        pltpu.VMEM((NBLK, SB * K), jnp.int32),
        pltpu.VMEM((SB * K, C), jnp.bfloat16),
        pltpu.VMEM((SB * K, C), jnp.bfloat16),
        pltpu.VMEM((SB, M * C), jnp.bfloat16),
        pltpu.VMEM((SB, M * C), jnp.bfloat16),
        pltpu.SemaphoreType.DMA,
        pltpu.SemaphoreType.DMA,
        pltpu.SemaphoreType.DMA,
        pltpu.SemaphoreType.DMA,
    ],
    compiler_params=pltpu.CompilerParams(
        needs_layout_passes=False, use_tc_tiling_on_sc=False),
)


def _mm_body(a_ref, w_ref, o_ref):
    o_ref[...] = jnp.dot(a_ref[...], w_ref[...],
                         preferred_element_type=jnp.float32)


def _mm_bias_body(a_ref, w_ref, b_ref, o_ref):
    o_ref[...] = jnp.dot(a_ref[...], w_ref[...],
                         preferred_element_type=jnp.float32) + b_ref[...]


def _matmul(a, w, blk):
    n, kk = a.shape
    _, out = w.shape
    return pl.pallas_call(
        _mm_body,
        grid=(n // blk,),
        in_specs=[
            pl.BlockSpec((blk, kk), lambda i: (i, 0)),
            pl.BlockSpec((kk, out), lambda i: (0, 0)),
        ],
        out_specs=pl.BlockSpec((blk, out), lambda i: (i, 0)),
        out_shape=jax.ShapeDtypeStruct((n, out), jnp.float32),
    )(a, w)


def _matmul_bias(a, w, bias, blk):
    n, kk = a.shape
    _, out = w.shape
    return pl.pallas_call(
        _mm_bias_body,
        grid=(n // blk,),
        in_specs=[
            pl.BlockSpec((blk, kk), lambda i: (i, 0)),
            pl.BlockSpec((kk, out), lambda i: (0, 0)),
            pl.BlockSpec((1, out), lambda i: (0, 0)),
        ],
        out_specs=pl.BlockSpec((blk, out), lambda i: (i, 0)),
        out_shape=jax.ShapeDtypeStruct((n, out), jnp.float32),
    )(a, w, bias)


def kernel(x, adj, W, b, u, c):
    u_padT = jnp.zeros((C, 128), jnp.float32).at[:, :M].set(u.T)
    ux = _matmul(x, u_padT, blk=2000)[:, :M]
    x_tab = (jnp.zeros((TPAD, C), jnp.bfloat16)
             .at[1:N + 1].set(x.astype(jnp.bfloat16)))
    ux_tab = jnp.zeros((TPAD, M), jnp.float32).at[1:N + 1].set(ux)
    uxc_pad = jnp.zeros((NPAD, M), jnp.float32).at[:N].set(ux + c[None, :])
    adj_pad = jnp.zeros((NPAD, K), jnp.int32).at[:N].set(adj)
    adj_blk = adj_pad.reshape(NPAD // SB, SB * K)
    y = _sc_call(x_tab, adj_blk, ux_tab, uxc_pad)
    Wcat = jnp.transpose(W, (0, 2, 1)).reshape(M * C, OUT)
    return _matmul_bias(y[:N], Wcat.astype(jnp.bfloat16),
                        b.reshape(1, OUT), blk=2000)


# speedup vs baseline: 1.5378x; 1.0779x over previous
import jax
import jax.numpy as jnp
from jax import lax
from jax.experimental import pallas as pl
from jax.experimental.pallas import tpu as pltpu
from jax.experimental.pallas import tpu_sc as plsc

N, C, K, M, OUT = 10000, 128, 16, 4, 128
NUM_WORKERS = 32
NB = 320
NPAD = NUM_WORKERS * NB
TPAD = 10016
L = 16
CCH = C // L


SB = 8
NBLK = NB // SB
LB = 2 * L
CHB = C // LB


def _sc_aggregate(x_hbm, adjg_hbm, adj_hbm, uxt_hbm, uxc_hbm, y_hbm,
                  ux_v, uxc_v, adj_v, adjg_v, rows0_v, rows1_v, st0_v, st1_v,
                  gsem0, gsem1, osem0, osem1):
    nc = plsc.get_sparse_core_info().num_cores
    wid = lax.axis_index("s") * nc + lax.axis_index("c")
    base = wid * NB
    pltpu.sync_copy(uxt_hbm, ux_v)
    pltpu.sync_copy(adj_hbm.at[pl.ds(wid * NBLK, NBLK)], adj_v)
    pltpu.sync_copy(adjg_hbm.at[pl.ds(wid * NBLK, NBLK)], adjg_v)
    pltpu.sync_copy(uxc_hbm.at[pl.ds(base, NB)], uxc_v)

    midx = [jnp.full((L,), m, jnp.int32) for m in range(M)]

    def process(jb, rows_ref, st_ref):
        def node(s, carry):
            adj_vec = adj_v[jb, pl.ds(s * K, K)]
            nid = jnp.full((L,), jb * SB + s, jnp.int32)
            nb_logit = [plsc.load_gather(ux_v, [adj_vec, midx[m]])
                        for m in range(M)]
            own = [plsc.load_gather(uxc_v, [nid, midx[m]]) for m in range(M)]
            p = [own[m] - nb_logit[m] for m in range(M)]
            pmax = jnp.maximum(jnp.maximum(p[0], p[1]),
                               jnp.maximum(p[2], p[3]))
            e = [jnp.exp(pm - pmax) for pm in p]
            ssum = e[0] + e[1] + e[2] + e[3]
            mask = adj_vec != 0
            cnt = plsc.all_reduce_population_count(mask)
            inv = jnp.where(cnt > 0, 1.0 / cnt.astype(jnp.float32), 0.0)
            scale = inv / ssum
            zero = jnp.zeros((L,), jnp.float32)
            wv = [jnp.where(mask, e[m] * scale, zero) for m in range(M)]
            acc = [[jnp.zeros((LB,), jnp.bfloat16) for _ in range(CHB)]
                   for _ in range(M)]
            for k in range(K):
                r = [rows_ref[s * K + k, cc * LB:(cc + 1) * LB]
                     for cc in range(CHB)]
                for m in range(M):
                    wk16 = jnp.full((L,), wv[m][k], jnp.float32)
                    wkb = plsc.pack(wk16, wk16,
                                    format=plsc.PackFormat.INTERLEAVED)
                    for cc in range(CHB):
                        acc[m][cc] = acc[m][cc] + wkb * r[cc]
            for m in range(M):
                for cc in range(CHB):
                    st_ref[s, m * C + cc * LB:m * C + (cc + 1) * LB] = \
                        acc[m][cc]
            return carry

        lax.fori_loop(0, SB, node, 0)

    def flush(jb, st_ref, osem):
        pltpu.async_copy(st_ref, y_hbm.at[pl.ds(base + jb * SB, SB)], osem)

    def gather(jb, rows_ref, gsem):
        pltpu.async_copy(x_hbm.at[adjg_v.at[jb]], rows_ref, gsem)

    gather(0, rows0_v, gsem0)

    def body(t, carry):
        jb0 = 2 * t
        jb1 = 2 * t + 1
        gather(jb1, rows1_v, gsem1)
        pltpu.make_async_copy(x_hbm.at[adjg_v.at[jb0]], rows0_v, gsem0).wait()

        @pl.when(t > 0)
        def _():
            pltpu.make_async_copy(
                st0_v, y_hbm.at[pl.ds(base, SB)], osem0).wait()

        process(jb0, rows0_v, st0_v)
        flush(jb0, st0_v, osem0)
        gather(jnp.minimum(jb0 + 2, NBLK - 1), rows0_v, gsem0)
        pltpu.make_async_copy(x_hbm.at[adjg_v.at[jb1]], rows1_v, gsem1).wait()

        @pl.when(t > 0)
        def _():
            pltpu.make_async_copy(
                st1_v, y_hbm.at[pl.ds(base, SB)], osem1).wait()

        process(jb1, rows1_v, st1_v)
        flush(jb1, st1_v, osem1)
        return carry

    lax.fori_loop(0, NBLK // 2, body, 0)
    pltpu.make_async_copy(x_hbm.at[adjg_v.at[NBLK - 1]], rows0_v, gsem0).wait()
    pltpu.make_async_copy(st0_v, y_hbm.at[pl.ds(base, SB)], osem0).wait()
    pltpu.make_async_copy(st1_v, y_hbm.at[pl.ds(base, SB)], osem1).wait()


_sc_call = pl.kernel(
    _sc_aggregate,
    out_type=jax.ShapeDtypeStruct((NPAD, M * C), jnp.bfloat16),
    mesh=plsc.VectorSubcoreMesh(core_axis_name="c", subcore_axis_name="s"),
    scratch_types=[
        pltpu.VMEM((TPAD, M), jnp.float32),
        pltpu.VMEM((NB, M), jnp.float32),
        pltpu.VMEM((NBLK, SB * K), jnp.int32),
        pltpu.VMEM((NBLK, SB * K), jnp.int32),
        pltpu.VMEM((SB * K, C), jnp.bfloat16),
        pltpu.VMEM((SB * K, C), jnp.bfloat16),
        pltpu.VMEM((SB, M * C), jnp.bfloat16),
        pltpu.VMEM((SB, M * C), jnp.bfloat16),
        pltpu.SemaphoreType.DMA,
        pltpu.SemaphoreType.DMA,
        pltpu.SemaphoreType.DMA,
        pltpu.SemaphoreType.DMA,
    ],
    compiler_params=pltpu.CompilerParams(
        needs_layout_passes=False, use_tc_tiling_on_sc=False),
)


def _pre_body(x_ref, u_ref, ux_ref, xb_ref):
    xv = x_ref[...]
    ux_ref[...] = jnp.dot(xv, u_ref[...],
                          preferred_element_type=jnp.float32)
    xb_ref[...] = xv.astype(jnp.bfloat16)


def _pre(x, u_padT, blk):
    n, kk = x.shape
    return pl.pallas_call(
        _pre_body,
        grid=(n // blk,),
        in_specs=[
            pl.BlockSpec((blk, kk), lambda i: (i, 0)),
            pl.BlockSpec((kk, 128), lambda i: (0, 0)),
        ],
        out_specs=[
            pl.BlockSpec((blk, 128), lambda i: (i, 0)),
            pl.BlockSpec((blk, kk), lambda i: (i, 0)),
        ],
        out_shape=[
            jax.ShapeDtypeStruct((n, 128), jnp.float32),
            jax.ShapeDtypeStruct((n, kk), jnp.bfloat16),
        ],
    )(x, u_padT)


def _mm_bias_body(a_ref, w_ref, b_ref, o_ref):
    o_ref[...] = jnp.dot(a_ref[...], w_ref[...],
                         preferred_element_type=jnp.float32) + b_ref[...]


def _matmul_bias(a, w, bias, blk, nout):
    n, kk = a.shape
    _, out = w.shape
    return pl.pallas_call(
        _mm_bias_body,
        grid=(n // blk,),
        in_specs=[
            pl.BlockSpec((blk, kk), lambda i: (i, 0)),
            pl.BlockSpec((kk, out), lambda i: (0, 0)),
            pl.BlockSpec((1, out), lambda i: (0, 0)),
        ],
        out_specs=pl.BlockSpec((blk, out), lambda i: (i, 0)),
        out_shape=jax.ShapeDtypeStruct((nout, out), jnp.float32),
    )(a, w, bias)


def kernel(x, adj, W, b, u, c):
    u_padT = jnp.zeros((C, 128), jnp.float32).at[:, :M].set(u.T)
    ux_full, x_tab = _pre(x, u_padT, blk=2000)
    ux = ux_full[:, :M]
    ux_tab = jnp.zeros((TPAD, M), jnp.float32).at[1:N + 1].set(ux)
    uxc_pad = jnp.zeros((NPAD, M), jnp.float32).at[:N].set(ux + c[None, :])
    adj_pad = jnp.zeros((NPAD, K), jnp.int32).at[:N].set(adj)
    adj_blk = adj_pad.reshape(NPAD // SB, SB * K)
    adjg_blk = jnp.maximum(adj_blk - 1, 0)
    y = _sc_call(x_tab, adjg_blk, adj_blk, ux_tab, uxc_pad)
    Wcat = jnp.transpose(W, (0, 2, 1)).reshape(M * C, OUT)
    return _matmul_bias(y, Wcat.astype(jnp.bfloat16),
                        b.reshape(1, OUT), blk=2048, nout=N)


# speedup vs baseline: 1.9131x; 1.2441x over previous
import jax
import jax.numpy as jnp
from jax import lax
from jax.experimental import pallas as pl
from jax.experimental.pallas import tpu as pltpu
from jax.experimental.pallas import tpu_sc as plsc

N, C, K, M, OUT = 10000, 128, 16, 4, 128
NUM_WORKERS = 32
NB = 320
NPAD = NUM_WORKERS * NB
TPAD = 10016
L = 16
CCH = C // L


SB = 8
NBLK = NB // SB
LB = 2 * L
CHB = C // LB


def _sc_aggregate(x_hbm, adjg_hbm, adj_hbm, uxt_hbm, uxc_hbm, y_hbm,
                  ux_v, uxc_v, adj_v, adjg_v, rows0_v, rows1_v, st0_v, st1_v,
                  gsem0, gsem1, osem0, osem1):
    nc = plsc.get_sparse_core_info().num_cores
    wid = lax.axis_index("s") * nc + lax.axis_index("c")
    base = wid * NB
    pltpu.sync_copy(uxt_hbm, ux_v)
    pltpu.sync_copy(adj_hbm.at[pl.ds(wid * NBLK, NBLK)], adj_v)
    pltpu.sync_copy(adjg_hbm.at[pl.ds(wid * NBLK, NBLK)], adjg_v)
    pltpu.sync_copy(uxc_hbm.at[pl.ds(base, NB)], uxc_v)

    midx = [jnp.full((L,), m, jnp.int32) for m in range(M)]

    def process(jb, rows_ref, st_ref):
        def node(s, carry):
            adj_vec = adj_v[jb, pl.ds(s * K, K)]
            nid = jnp.full((L,), jb * SB + s, jnp.int32)
            nb_logit = [plsc.load_gather(ux_v, [adj_vec, midx[m]])
                        for m in range(M)]
            own = [plsc.load_gather(uxc_v, [nid, midx[m]]) for m in range(M)]
            p = [own[m] - nb_logit[m] for m in range(M)]
            pmax = jnp.maximum(jnp.maximum(p[0], p[1]),
                               jnp.maximum(p[2], p[3]))
            e = [jnp.exp(pm - pmax) for pm in p]
            ssum = e[0] + e[1] + e[2] + e[3]
            mask = adj_vec != 0
            cnt = plsc.all_reduce_population_count(mask)
            inv = jnp.where(cnt > 0, 1.0 / cnt.astype(jnp.float32), 0.0)
            scale = inv / ssum
            zero = jnp.zeros((L,), jnp.float32)
            wv = [jnp.where(mask, e[m] * scale, zero) for m in range(M)]
            acc = [[jnp.zeros((LB,), jnp.bfloat16) for _ in range(CHB)]
                   for _ in range(M)]
            for k in range(K):
                r = [rows_ref[s * K + k, cc * LB:(cc + 1) * LB]
                     for cc in range(CHB)]
                for m in range(M):
                    wk16 = jnp.full((L,), wv[m][k], jnp.float32)
                    wkb = plsc.pack(wk16, wk16,
                                    format=plsc.PackFormat.INTERLEAVED)
                    for cc in range(CHB):
                        acc[m][cc] = acc[m][cc] + wkb * r[cc]
            for m in range(M):
                for cc in range(CHB):
                    ev, od = plsc.unpack(acc[m][cc],
                                         format=plsc.PackFormat.INTERLEAVED)
                    st_ref[m, s, cc * LB:cc * LB + L] = ev
                    st_ref[m, s, cc * LB + L:(cc + 1) * LB] = od
            return carry

        lax.fori_loop(0, SB, node, 0)

    def flush(jb, st_ref, osem):
        pltpu.async_copy(st_ref, y_hbm.at[wid * NBLK + jb], osem)

    def gather(jb, rows_ref, gsem):
        pltpu.async_copy(x_hbm.at[adjg_v.at[jb]], rows_ref, gsem)

    gather(0, rows0_v, gsem0)

    def body(t, carry):
        jb0 = 2 * t
        jb1 = 2 * t + 1
        gather(jb1, rows1_v, gsem1)
        pltpu.make_async_copy(x_hbm.at[adjg_v.at[jb0]], rows0_v, gsem0).wait()

        @pl.when(t > 0)
        def _():
            pltpu.make_async_copy(st0_v, y_hbm.at[0], osem0).wait()

        process(jb0, rows0_v, st0_v)
        flush(jb0, st0_v, osem0)
        gather(jnp.minimum(jb0 + 2, NBLK - 1), rows0_v, gsem0)
        pltpu.make_async_copy(x_hbm.at[adjg_v.at[jb1]], rows1_v, gsem1).wait()

        @pl.when(t > 0)
        def _():
            pltpu.make_async_copy(st1_v, y_hbm.at[0], osem1).wait()

        process(jb1, rows1_v, st1_v)
        flush(jb1, st1_v, osem1)
        return carry

    lax.fori_loop(0, NBLK // 2, body, 0)
    pltpu.make_async_copy(x_hbm.at[adjg_v.at[NBLK - 1]], rows0_v, gsem0).wait()
    pltpu.make_async_copy(st0_v, y_hbm.at[0], osem0).wait()
    pltpu.make_async_copy(st1_v, y_hbm.at[0], osem1).wait()


_sc_call = pl.kernel(
    _sc_aggregate,
    out_type=jax.ShapeDtypeStruct((NPAD // SB, M, SB, C), jnp.float32),
    mesh=plsc.VectorSubcoreMesh(core_axis_name="c", subcore_axis_name="s"),
    scratch_types=[
        pltpu.VMEM((TPAD, M), jnp.float32),
        pltpu.VMEM((NB, M), jnp.float32),
        pltpu.VMEM((NBLK, SB * K), jnp.int32),
        pltpu.VMEM((NBLK, SB * K), jnp.int32),
        pltpu.VMEM((SB * K, C), jnp.bfloat16),
        pltpu.VMEM((SB * K, C), jnp.bfloat16),
        pltpu.VMEM((M, SB, C), jnp.float32),
        pltpu.VMEM((M, SB, C), jnp.float32),
        pltpu.SemaphoreType.DMA,
        pltpu.SemaphoreType.DMA,
        pltpu.SemaphoreType.DMA,
        pltpu.SemaphoreType.DMA,
    ],
    compiler_params=pltpu.CompilerParams(
        needs_layout_passes=False, use_tc_tiling_on_sc=False),
)


def _pre_body(x_ref, u_ref, ux_ref, xb_ref):
    xv = x_ref[...]
    ux_ref[...] = jnp.dot(xv, u_ref[...],
                          preferred_element_type=jnp.float32)
    xb_ref[...] = xv.astype(jnp.bfloat16)


def _pre(x, u_padT, blk):
    n, kk = x.shape
    return pl.pallas_call(
        _pre_body,
        grid=(n // blk,),
        in_specs=[
            pl.BlockSpec((blk, kk), lambda i: (i, 0)),
            pl.BlockSpec((kk, 128), lambda i: (0, 0)),
        ],
        out_specs=[
            pl.BlockSpec((blk, 128), lambda i: (i, 0)),
            pl.BlockSpec((blk, kk), lambda i: (i, 0)),
        ],
        out_shape=[
            jax.ShapeDtypeStruct((n, 128), jnp.float32),
            jax.ShapeDtypeStruct((n, kk), jnp.bfloat16),
        ],
    )(x, u_padT)


def _mm_bias_body(a_ref, w_ref, b_ref, o_ref):
    acc = None
    for m in range(M):
        am = a_ref[:, m, :, :].reshape(-1, C).astype(jnp.bfloat16)
        d = jnp.dot(am, w_ref[m], preferred_element_type=jnp.float32)
        acc = d if acc is None else acc + d
    o_ref[...] = acc + b_ref[...]


def _matmul_bias(a, w, bias, bj, nout):
    nb = a.shape[0]
    out = w.shape[2]
    return pl.pallas_call(
        _mm_bias_body,
        grid=(nb // bj,),
        in_specs=[
            pl.BlockSpec((bj, M, SB, C), lambda i: (i, 0, 0, 0)),
            pl.BlockSpec((M, C, out), lambda i: (0, 0, 0)),
            pl.BlockSpec((1, out), lambda i: (0, 0)),
        ],
        out_specs=pl.BlockSpec((bj * SB, out), lambda i: (i, 0)),
        out_shape=jax.ShapeDtypeStruct((nout, out), jnp.float32),
    )(a, w, bias)


def kernel(x, adj, W, b, u, c):
    u_padT = jnp.zeros((C, 128), jnp.float32).at[:, :M].set(u.T)
    ux_full, x_tab = _pre(x, u_padT, blk=2000)
    ux = ux_full[:, :M]
    ux_tab = jnp.zeros((TPAD, M), jnp.float32).at[1:N + 1].set(ux)
    uxc_pad = jnp.zeros((NPAD, M), jnp.float32).at[:N].set(ux + c[None, :])
    adj_pad = jnp.zeros((NPAD, K), jnp.int32).at[:N].set(adj)
    adj_blk = adj_pad.reshape(NPAD // SB, SB * K)
    adjg_blk = jnp.maximum(adj_blk - 1, 0)
    y = _sc_call(x_tab, adjg_blk, adj_blk, ux_tab, uxc_pad)
    chmap = jnp.array([32 * (r // 32)
                       + (2 * (r % 32) if r % 32 < 16
                          else 2 * (r % 32 - 16) + 1)
                       for r in range(C)], dtype=jnp.int32)
    Wperm = jnp.transpose(W, (0, 2, 1))[:, chmap, :]
    return _matmul_bias(y, Wperm.astype(jnp.bfloat16),
                        b.reshape(1, OUT), bj=256, nout=N)
